# Initial kernel scaffold; baseline (speedup 1.0000x reference)
import jax
import jax.numpy as jnp
from jax.experimental import pallas as pl


def kernel(pc_source, pc_target, pred_flow):
    raise NotImplementedError("write your pallas kernel here")


# speedup vs baseline: 94.0207x; 94.0207x over previous
import functools

import jax
import jax.numpy as jnp
from jax.experimental import pallas as pl
from jax.experimental.pallas import tpu as pltpu

_QB = 512


def _chamfer_body(q_ref, r_ref, out_ref, colmin_ref, colex_ref, *, nq, inv_n, inv_m):
    b = pl.program_id(0)
    i = pl.program_id(1)

    q = q_ref[0]
    r = r_ref[0]
    QB, M = q.shape[0], r.shape[1]

    qr = jax.lax.dot_general(
        q,
        r,
        (((1,), (0,)), ((), ())),
        preferred_element_type=jnp.float32,
    )
    qn = jnp.sum(q * q, axis=1, keepdims=True)
    rn = jnp.sum(r * r, axis=0, keepdims=True)
    d2a = (qn + rn) - 2.0 * qr

    dx = q[:, 0:1] - r[0:1, :]
    dy = q[:, 1:2] - r[1:2, :]
    dz = q[:, 2:3] - r[2:3, :]
    d2e = dx * dx + dy * dy + dz * dz

    iota_m = jax.lax.broadcasted_iota(jnp.int32, (QB, M), 1)
    iota_n = jax.lax.broadcasted_iota(jnp.int32, (QB, M), 0)

    rowmin = jnp.min(d2a, axis=1, keepdims=True)
    midx = jnp.min(
        jnp.where(d2a == rowmin, iota_m, M), axis=1, keepdims=True
    )
    ex1 = jnp.sum(
        jnp.where(iota_m == midx, d2e, 0.0), axis=1, keepdims=True
    )
    part = jnp.sum(jnp.sqrt(jnp.maximum(ex1, 0.0)), keepdims=True)

    colmin_t = jnp.min(d2a, axis=0, keepdims=True)
    nidx = jnp.min(
        jnp.where(d2a == colmin_t, iota_n, QB), axis=0, keepdims=True
    )
    colex_t = jnp.sum(
        jnp.where(iota_n == nidx, d2e, 0.0), axis=0, keepdims=True
    )

    @pl.when(jnp.logical_and(b == 0, i == 0))
    def _init():
        out_ref[...] = jnp.zeros((1, 1), jnp.float32)

    @pl.when(i == 0)
    def _first():
        colmin_ref[...] = colmin_t
        colex_ref[...] = colex_t

    @pl.when(i != 0)
    def _rest():
        repl = colmin_t < colmin_ref[...]
        colex_ref[...] = jnp.where(repl, colex_t, colex_ref[...])
        colmin_ref[...] = jnp.where(repl, colmin_t, colmin_ref[...])

    out_ref[...] += part * inv_n

    @pl.when(i == nq - 1)
    def _last():
        out_ref[...] += (
            jnp.sum(
                jnp.sqrt(jnp.maximum(colex_ref[...], 0.0)), keepdims=True
            )
            * inv_m
        )


@jax.jit
def kernel(pc_source, pc_target, pred_flow):
    B, N, _ = pc_source.shape
    M = pc_target.shape[1]

    pred = pc_source + pred_flow
    raT = jnp.transpose(pc_target, (0, 2, 1))

    nq = N // _QB
    out = pl.pallas_call(
        functools.partial(
            _chamfer_body, nq=nq, inv_n=1.0 / (B * N), inv_m=1.0 / (B * M)
        ),
        grid=(B, nq),
        in_specs=[
            pl.BlockSpec((1, _QB, 3), lambda b, i: (b, i, 0)),
            pl.BlockSpec((1, 3, M), lambda b, i: (b, 0, 0)),
        ],
        out_specs=pl.BlockSpec((1, 1), lambda b, i: (0, 0)),
        out_shape=jax.ShapeDtypeStruct((1, 1), jnp.float32),
        scratch_shapes=[
            pltpu.VMEM((1, M), jnp.float32),
            pltpu.VMEM((1, M), jnp.float32),
        ],
    )(pred, raT)
    return out[0, 0]


# speedup vs baseline: 121.8279x; 1.2958x over previous
import functools

import jax
import jax.numpy as jnp
from jax.experimental import pallas as pl
from jax.experimental.pallas import tpu as pltpu

_QB = 512


def _chamfer_body(
    q_ref, qs_ref, r_ref, qn_ref, rn_ref, out_ref, colmin_ref, colex_ref,
    *, nq, inv_n, inv_m
):
    b = pl.program_id(0)
    i = pl.program_id(1)

    q = q_ref[0]
    qs = qs_ref[0]
    r = r_ref[0]
    qn = qn_ref[0]
    rn = rn_ref[0]

    qr2 = jax.lax.dot_general(
        qs,
        r,
        (((1,), (0,)), ((), ())),
        preferred_element_type=jnp.float32,
    )
    d2a = (qn + rn) + qr2

    dx = q[:, 0:1] - r[0:1, :]
    dy = q[:, 1:2] - r[1:2, :]
    dz = q[:, 2:3] - r[2:3, :]
    d2e = dx * dx + dy * dy + dz * dz

    inf = jnp.float32(jnp.inf)

    rowmin = jnp.min(d2a, axis=1, keepdims=True)
    ex1 = jnp.min(
        jnp.where(d2a == rowmin, d2e, inf), axis=1, keepdims=True
    )
    part = jnp.sum(jnp.sqrt(ex1), keepdims=True)

    colmin_t = jnp.min(d2a, axis=0, keepdims=True)
    colex_t = jnp.min(
        jnp.where(d2a == colmin_t, d2e, inf), axis=0, keepdims=True
    )

    @pl.when(jnp.logical_and(b == 0, i == 0))
    def _init():
        out_ref[...] = jnp.zeros((1, 1), jnp.float32)

    @pl.when(i == 0)
    def _first():
        colmin_ref[...] = colmin_t
        colex_ref[...] = colex_t

    @pl.when(i != 0)
    def _rest():
        repl = colmin_t < colmin_ref[...]
        colex_ref[...] = jnp.where(repl, colex_t, colex_ref[...])
        colmin_ref[...] = jnp.where(repl, colmin_t, colmin_ref[...])

    out_ref[...] += part * inv_n

    @pl.when(i == nq - 1)
    def _last():
        out_ref[...] += jnp.sum(jnp.sqrt(colex_ref[...]), keepdims=True) * inv_m


@jax.jit
def kernel(pc_source, pc_target, pred_flow):
    B, N, _ = pc_source.shape
    M = pc_target.shape[1]

    pred = pc_source + pred_flow
    pred_s = -2.0 * pred
    raT = jnp.transpose(pc_target, (0, 2, 1))
    qn = jnp.sum(pred * pred, axis=-1, keepdims=True)
    rn = jnp.sum(pc_target * pc_target, axis=-1)[:, None, :]

    nq = N // _QB
    out = pl.pallas_call(
        functools.partial(
            _chamfer_body, nq=nq, inv_n=1.0 / (B * N), inv_m=1.0 / (B * M)
        ),
        grid=(B, nq),
        in_specs=[
            pl.BlockSpec((1, _QB, 3), lambda b, i: (b, i, 0)),
            pl.BlockSpec((1, _QB, 3), lambda b, i: (b, i, 0)),
            pl.BlockSpec((1, 3, M), lambda b, i: (b, 0, 0)),
            pl.BlockSpec((1, _QB, 1), lambda b, i: (b, i, 0)),
            pl.BlockSpec((1, 1, M), lambda b, i: (b, 0, 0)),
        ],
        out_specs=pl.BlockSpec((1, 1), lambda b, i: (0, 0)),
        out_shape=jax.ShapeDtypeStruct((1, 1), jnp.float32),
        scratch_shapes=[
            pltpu.VMEM((1, M), jnp.float32),
            pltpu.VMEM((1, M), jnp.float32),
        ],
    )(pred, pred_s, raT, qn, rn)
    return out[0, 0]


# speedup vs baseline: 124.2697x; 1.0200x over previous
import functools

import jax
import jax.numpy as jnp
from jax.experimental import pallas as pl
from jax.experimental.pallas import tpu as pltpu

_QB = 1024


def _chamfer_body(
    q_ref, qs_ref, r_ref, qn_ref, rn_ref, out_ref, colmin_ref, colex_ref,
    *, nq, inv_n, inv_m
):
    b = pl.program_id(0)
    i = pl.program_id(1)

    q = q_ref[0]
    qs = qs_ref[0]
    r = r_ref[0]
    qn = qn_ref[0]
    rn = rn_ref[0]

    qr2 = jax.lax.dot_general(
        qs,
        r,
        (((1,), (0,)), ((), ())),
        preferred_element_type=jnp.float32,
    )
    d2a = (qn + rn) + qr2

    dx = q[:, 0:1] - r[0:1, :]
    dy = q[:, 1:2] - r[1:2, :]
    dz = q[:, 2:3] - r[2:3, :]
    d2e = dx * dx + dy * dy + dz * dz

    inf = jnp.float32(jnp.inf)

    rowmin = jnp.min(d2a, axis=1, keepdims=True)
    ex1 = jnp.min(
        jnp.where(d2a == rowmin, d2e, inf), axis=1, keepdims=True
    )
    part = jnp.sum(jnp.sqrt(ex1), keepdims=True)

    colmin_t = jnp.min(d2a, axis=0, keepdims=True)
    colex_t = jnp.min(
        jnp.where(d2a == colmin_t, d2e, inf), axis=0, keepdims=True
    )

    @pl.when(jnp.logical_and(b == 0, i == 0))
    def _init():
        out_ref[...] = jnp.zeros((1, 1), jnp.float32)

    @pl.when(i == 0)
    def _first():
        colmin_ref[...] = colmin_t
        colex_ref[...] = colex_t

    @pl.when(i != 0)
    def _rest():
        repl = colmin_t < colmin_ref[...]
        colex_ref[...] = jnp.where(repl, colex_t, colex_ref[...])
        colmin_ref[...] = jnp.where(repl, colmin_t, colmin_ref[...])

    out_ref[...] += part * inv_n

    @pl.when(i == nq - 1)
    def _last():
        out_ref[...] += jnp.sum(jnp.sqrt(colex_ref[...]), keepdims=True) * inv_m


@jax.jit
def kernel(pc_source, pc_target, pred_flow):
    B, N, _ = pc_source.shape
    M = pc_target.shape[1]

    pred = pc_source + pred_flow
    pred_s = -2.0 * pred
    raT = jnp.transpose(pc_target, (0, 2, 1))
    qn = jnp.sum(pred * pred, axis=-1, keepdims=True)
    rn = jnp.sum(pc_target * pc_target, axis=-1)[:, None, :]

    nq = N // _QB
    out = pl.pallas_call(
        functools.partial(
            _chamfer_body, nq=nq, inv_n=1.0 / (B * N), inv_m=1.0 / (B * M)
        ),
        grid=(B, nq),
        in_specs=[
            pl.BlockSpec((1, _QB, 3), lambda b, i: (b, i, 0)),
            pl.BlockSpec((1, _QB, 3), lambda b, i: (b, i, 0)),
            pl.BlockSpec((1, 3, M), lambda b, i: (b, 0, 0)),
            pl.BlockSpec((1, _QB, 1), lambda b, i: (b, i, 0)),
            pl.BlockSpec((1, 1, M), lambda b, i: (b, 0, 0)),
        ],
        out_specs=pl.BlockSpec((1, 1), lambda b, i: (0, 0)),
        out_shape=jax.ShapeDtypeStruct((1, 1), jnp.float32),
        scratch_shapes=[
            pltpu.VMEM((1, M), jnp.float32),
            pltpu.VMEM((1, M), jnp.float32),
        ],
    )(pred, pred_s, raT, qn, rn)
    return out[0, 0]


# speedup vs baseline: 150.5499x; 1.2115x over previous
import functools

import jax
import jax.numpy as jnp
from jax.experimental import pallas as pl
from jax.experimental.pallas import tpu as pltpu

_QB = 1024


def _chamfer_body(
    qs_ref, r_ref, qn_ref, rn_ref, out_ref, colmin_ref, colex_ref,
    *, nq, inv_n, inv_m
):
    b = pl.program_id(0)
    i = pl.program_id(1)

    qs = qs_ref[0]
    r = r_ref[0]
    qn = qn_ref[0]
    rn = rn_ref[0]

    qr2 = jax.lax.dot_general(
        qs,
        r,
        (((1,), (0,)), ((), ())),
        preferred_element_type=jnp.float32,
    )
    t = qn + rn
    d2a = t + qr2

    dot2 = (
        qs[:, 0:1] * r[0:1, :]
        + qs[:, 1:2] * r[1:2, :]
        + qs[:, 2:3] * r[2:3, :]
    )
    delta = dot2 - qr2

    inf = jnp.float32(jnp.inf)

    rowmin = jnp.min(d2a, axis=1, keepdims=True)
    drow = jnp.min(
        jnp.where(d2a == rowmin, delta, inf), axis=1, keepdims=True
    )
    ex1 = jnp.maximum(rowmin + drow, 0.0)
    part = jnp.sum(jnp.sqrt(ex1), keepdims=True)

    colmin_t = jnp.min(d2a, axis=0, keepdims=True)
    dcol = jnp.min(
        jnp.where(d2a == colmin_t, delta, inf), axis=0, keepdims=True
    )
    colex_t = jnp.maximum(colmin_t + dcol, 0.0)

    @pl.when(jnp.logical_and(b == 0, i == 0))
    def _init():
        out_ref[...] = jnp.zeros((1, 1), jnp.float32)

    @pl.when(i == 0)
    def _first():
        colmin_ref[...] = colmin_t
        colex_ref[...] = colex_t

    @pl.when(i != 0)
    def _rest():
        repl = colmin_t < colmin_ref[...]
        colex_ref[...] = jnp.where(repl, colex_t, colex_ref[...])
        colmin_ref[...] = jnp.where(repl, colmin_t, colmin_ref[...])

    out_ref[...] += part * inv_n

    @pl.when(i == nq - 1)
    def _last():
        out_ref[...] += jnp.sum(jnp.sqrt(colex_ref[...]), keepdims=True) * inv_m


@jax.jit
def kernel(pc_source, pc_target, pred_flow):
    B, N, _ = pc_source.shape
    M = pc_target.shape[1]

    pred = pc_source + pred_flow
    pred_s = -2.0 * pred
    raT = jnp.transpose(pc_target, (0, 2, 1))
    qn = jnp.sum(pred * pred, axis=-1, keepdims=True)
    rn = jnp.sum(pc_target * pc_target, axis=-1)[:, None, :]

    nq = N // _QB
    out = pl.pallas_call(
        functools.partial(
            _chamfer_body, nq=nq, inv_n=1.0 / (B * N), inv_m=1.0 / (B * M)
        ),
        grid=(B, nq),
        in_specs=[
            pl.BlockSpec((1, _QB, 3), lambda b, i: (b, i, 0)),
            pl.BlockSpec((1, 3, M), lambda b, i: (b, 0, 0)),
            pl.BlockSpec((1, _QB, 1), lambda b, i: (b, i, 0)),
            pl.BlockSpec((1, 1, M), lambda b, i: (b, 0, 0)),
        ],
        out_specs=pl.BlockSpec((1, 1), lambda b, i: (0, 0)),
        out_shape=jax.ShapeDtypeStruct((1, 1), jnp.float32),
        scratch_shapes=[
            pltpu.VMEM((1, M), jnp.float32),
            pltpu.VMEM((1, M), jnp.float32),
        ],
    )(pred_s, raT, qn, rn)
    return out[0, 0]
